# baseline (speedup 1.0000x reference)
import functools

import jax
import jax.numpy as jnp
from jax import lax
from jax.experimental import pallas as pl
from jax.experimental.pallas import tpu as pltpu
from jax.experimental.pallas import tpu_sc as plsc

BS = 16384
V = 100000
D = 64
IN = 4
H = 32

_INFO = plsc.get_sparse_core_info()
_NC, _NS, _L = _INFO.num_cores, _INFO.num_subcores, _INFO.num_lanes
_NW = _NC * _NS
_BPW = BS // _NW
_IDX_MINOR = 128
_NG = _BPW // _IDX_MINOR


def _sc_gather(joint_info, joint_token, emb):
    @functools.partial(
        pl.kernel,
        out_type=jax.ShapeDtypeStruct((BS, D), jnp.float32),
        mesh=plsc.VectorSubcoreMesh(core_axis_name="c", subcore_axis_name="s"),
        compiler_params=pltpu.CompilerParams(use_tc_tiling_on_sc=False),
        scratch_types=[
            pltpu.VMEM((_NG, _IDX_MINOR), jnp.int32),
            pltpu.VMEM((IN, _BPW), jnp.float32),
            pltpu.VMEM((_BPW, D), jnp.float32),
            pltpu.SemaphoreType.DMA,
        ],
    )
    def body(info_hbm, tok_hbm, emb_hbm, out_hbm, idx_v, info_v, rows_v, sem):
        wid = lax.axis_index("s") * _NC + lax.axis_index("c")
        base = wid * _BPW
        for j in range(_NG):
            pltpu.sync_copy(
                tok_hbm.at[pl.ds(base + j * _IDX_MINOR, _IDX_MINOR)],
                idx_v.at[j],
            )
        for i in range(IN):
            pltpu.sync_copy(info_hbm.at[i, pl.ds(base, _BPW)], info_v.at[i])

        for g in range(_BPW // _L):
            mask = None
            for i in range(IN):
                col = info_v[i, pl.ds(g * _L, _L)]
                zc = col == 0.0
                mask = zc if mask is None else (mask & zc)
            j, off = divmod(g * _L, _IDX_MINOR)
            tok16 = idx_v[j, pl.ds(off, _L)]
            idx_v[j, pl.ds(off, _L)] = jnp.where(mask, 0, tok16)

        copies = [
            pltpu.async_copy(
                emb_hbm.at[idx_v.at[j]],
                rows_v.at[pl.ds(j * _IDX_MINOR, _IDX_MINOR)],
                sem,
            )
            for j in range(_NG)
        ]
        for c in copies:
            c.wait()
        pltpu.sync_copy(rows_v, out_hbm.at[pl.ds(base, _BPW)])

    return body(joint_info, joint_token, emb)


_TC_ROWS = 2048


def _tc_body(info_ref, g_ref, w1t_ref, b1_ref, w2t_ref, o_ref):
    info = info_ref[...]
    h = jnp.dot(info, w1t_ref[...], preferred_element_type=jnp.float32)
    h = h + b1_ref[...]
    h = 0.5 * h * (1.0 + lax.erf(h * 0.7071067811865476))
    out = jnp.dot(h, w2t_ref[...], preferred_element_type=jnp.float32)
    mask = jnp.max(jnp.abs(info), axis=1, keepdims=True) == 0.0
    o_ref[...] = jnp.where(mask, 0.0, out) + g_ref[...]


def _tc_mlp_add(joint_info, gathered, W1, b1, W2):
    grid = BS // _TC_ROWS
    return pl.pallas_call(
        _tc_body,
        grid=(grid,),
        in_specs=[
            pl.BlockSpec((_TC_ROWS, IN), lambda i: (i, 0)),
            pl.BlockSpec((_TC_ROWS, D), lambda i: (i, 0)),
            pl.BlockSpec((IN, H), lambda i: (0, 0)),
            pl.BlockSpec((1, H), lambda i: (0, 0)),
            pl.BlockSpec((H, D), lambda i: (0, 0)),
        ],
        out_specs=pl.BlockSpec((_TC_ROWS, D), lambda i: (i, 0)),
        out_shape=jax.ShapeDtypeStruct((BS, D), jnp.float32),
    )(joint_info, gathered, W1.T, b1.reshape(1, H), W2.T)


def kernel(joint_info, joint_token, emb, W1, b1, W2):
    tok = joint_token.astype(jnp.int32)
    gathered = _sc_gather(joint_info.T, tok, emb)
    return _tc_mlp_add(joint_info, gathered, W1, b1, W2)
